# Initial kernel scaffold; baseline (speedup 1.0000x reference)
import jax
import jax.numpy as jnp
from jax.experimental import pallas as pl


def kernel(class_logits, box_regression, proposals):
    raise NotImplementedError("write your pallas kernel here")


# speedup vs baseline: 27.6285x; 27.6285x over previous
import functools

import jax
import jax.numpy as jnp
from jax.experimental import pallas as pl
from jax.experimental.pallas import tpu as pltpu

_N_BOXES = 20000
_NUM_CLASSES = 8
_SCORE_THRESH = 0.05
_NMS_THRESH = 0.5
_DETS = 100
_PRE_NMS = 2000
_M = 2048
_B = 256
_NB = _M // _B


def _eye(n):
    r = jax.lax.broadcasted_iota(jnp.int32, (n, n), 0)
    c = jax.lax.broadcasted_iota(jnp.int32, (n, n), 1)
    return (r == c).astype(jnp.float32)


def _to_col(v, eye):
    return jax.lax.dot_general(
        eye, v, (((1,), (1,)), ((), ())), preferred_element_type=jnp.float32
    )


def _iou_block(rc, cc):
    dx = jnp.clip(
        jnp.minimum(rc["maxx"], cc["maxx"]) - jnp.maximum(rc["minx"], cc["minx"]),
        0.0, None)
    dy = jnp.clip(
        jnp.minimum(rc["maxy"], cc["maxy"]) - jnp.maximum(rc["miny"], cc["miny"]),
        0.0, None)
    dz = jnp.clip(
        jnp.minimum(rc["maxz"], cc["maxz"]) - jnp.maximum(rc["minz"], cc["minz"]),
        0.0, None)
    inter = (dx * dy) * dz
    union = rc["vol"] + cc["vol"] - inter
    return inter / (union + 1e-6)


def _nms_kernel(reg_ref, prop_ref, s_ref, so_ref, bo_ref, a_scr):
    reg = reg_ref[0]
    prop = prop_ref[0]
    s = s_ref[0]

    px = prop[0:1]
    py = prop[1:2]
    pz = prop[2:3]
    pw = prop[3:4] + 0.05
    plen = prop[4:5] + 0.05
    ph = prop[5:6] + 0.05
    pyaw = prop[6:7]
    cx = reg[0:1] / 10.0 * pw + px
    cy = reg[1:2] / 10.0 * plen + py
    cz = reg[2:3] / 10.0 * ph + pz
    w = jnp.exp(jnp.clip(reg[3:4] / 5.0, -4.0, 4.0)) * pw
    l = jnp.exp(jnp.clip(reg[4:5] / 5.0, -4.0, 4.0)) * plen
    h = jnp.exp(jnp.clip(reg[5:6] / 5.0, -4.0, 4.0)) * ph
    yaw = reg[6:7] + pyaw

    bo_ref[0] = jnp.concatenate([cx, cy, cz, w, l, h, yaw], axis=0)

    row = {
        "minx": cx - w * 0.5, "miny": cy - l * 0.5, "minz": cz - h * 0.5,
        "maxx": cx + w * 0.5, "maxy": cy + l * 0.5, "maxz": cz + h * 0.5,
        "vol": (w * l) * h,
    }

    eye = _eye(_B)
    lane = jax.lax.broadcasted_iota(jnp.int32, (1, _B), 1)

    col_hist = {k: [] for k in row}
    keep_cols = []
    kb_rows = []
    for b in range(_NB):
        sl = slice(b * _B, (b + 1) * _B)
        blk = {k: v[:, sl] for k, v in row.items()}
        blk_col = {k: _to_col(v, eye) for k, v in blk.items()}
        valid = (s[:, sl] > 0.0).astype(jnp.float32)

        if b > 0:
            prev = {k: jnp.concatenate(col_hist[k], axis=0) for k in row}
            keep_prev = jnp.concatenate(keep_cols, axis=0)
            iou_prev = _iou_block(prev, blk)
            ext = jnp.any(
                jnp.logical_and(iou_prev > _NMS_THRESH, keep_prev > 0.5),
                axis=0, keepdims=True)
            valid_eff = jnp.where(ext, 0.0, valid)
        else:
            valid_eff = valid

        for k in row:
            col_hist[k].append(blk_col[k])

        a_scr[...] = _iou_block(blk_col, blk)

        def body(i, kb, valid_eff=valid_eff):
            arow = a_scr[pl.ds(i, 1), :]
            sup = jnp.any(jnp.logical_and(arow > _NMS_THRESH, kb > 0.5))
            upd = jnp.where(sup, 0.0, valid_eff)
            return jnp.where(lane == i, upd, kb)

        kb = jax.lax.fori_loop(0, _B, body, jnp.zeros((1, _B), jnp.float32))
        keep_cols.append(_to_col(kb, eye))
        kb_rows.append(kb)

    keep = jnp.concatenate(kb_rows, axis=1)
    so_ref[0] = jnp.where(keep > 0.5, s, -1.0)


@functools.partial(jax.jit, static_argnames=("interpret",))
def _run(class_logits, box_regression, proposals, interpret=False):
    n, c = class_logits.shape
    probs = jax.nn.softmax(class_logits, axis=-1)
    sj = jnp.where(probs[:, 1:] > _SCORE_THRESH, probs[:, 1:], -1.0).T
    top_s, top_i = jax.lax.top_k(sj, _PRE_NMS)

    reg3 = box_regression.reshape(n, c, 7)
    cls_ids = jnp.arange(1, c, dtype=jnp.int32)
    reg_cand = jax.vmap(lambda idx, j: reg3[idx, j])(top_i, cls_ids)
    prop_cand = proposals[top_i]

    pad = _M - _PRE_NMS
    s_p = jnp.pad(top_s, ((0, 0), (0, pad)), constant_values=-1.0)[:, None, :]
    regs_t = jnp.pad(reg_cand, ((0, 0), (0, pad), (0, 0))).transpose(0, 2, 1)
    props_t = jnp.pad(prop_cand, ((0, 0), (0, pad), (0, 0))).transpose(0, 2, 1)

    nc = c - 1
    scores_o, boxes_o = pl.pallas_call(
        _nms_kernel,
        grid=(nc,),
        in_specs=[
            pl.BlockSpec((1, 7, _M), lambda j: (j, 0, 0)),
            pl.BlockSpec((1, 7, _M), lambda j: (j, 0, 0)),
            pl.BlockSpec((1, 1, _M), lambda j: (j, 0, 0)),
        ],
        out_specs=[
            pl.BlockSpec((1, 1, _M), lambda j: (j, 0, 0)),
            pl.BlockSpec((1, 7, _M), lambda j: (j, 0, 0)),
        ],
        out_shape=[
            jax.ShapeDtypeStruct((nc, 1, _M), jnp.float32),
            jax.ShapeDtypeStruct((nc, 7, _M), jnp.float32),
        ],
        scratch_shapes=[pltpu.VMEM((_B, _B), jnp.float32)],
        interpret=interpret,
    )(regs_t, props_t, s_p)

    scores = scores_o[:, 0, :_PRE_NMS].reshape(-1)
    boxes = boxes_o.transpose(0, 2, 1)[:, :_PRE_NMS].reshape(-1, 7)
    labels = jnp.repeat(jnp.arange(1, c, dtype=jnp.int32), _PRE_NMS)

    fs, fi = jax.lax.top_k(scores, _DETS)
    fb = boxes[fi]
    fl = labels[fi]
    out = jnp.concatenate([fb, fs[:, None]], axis=1)
    return out, fl


def kernel(class_logits, box_regression, proposals):
    return _run(class_logits, box_regression, proposals)


# speedup vs baseline: 115.5901x; 4.1837x over previous
import functools

import jax
import jax.numpy as jnp
from jax.experimental import pallas as pl
from jax.experimental.pallas import tpu as pltpu

_N_BOXES = 20000
_NUM_CLASSES = 8
_SCORE_THRESH = 0.05
_NMS_THRESH = 0.5
_DETS = 100
_PRE_NMS = 2000
_M = 2048
_B = 256
_NB = _M // _B
_NC = _NUM_CLASSES - 1


def _eyeb(n):
    r = jax.lax.broadcasted_iota(jnp.int32, (_NC, n, n), 1)
    c = jax.lax.broadcasted_iota(jnp.int32, (_NC, n, n), 2)
    return (r == c).astype(jnp.float32)


def _to_col(v, eyeb):
    return jax.lax.dot_general(
        eyeb, v, (((2,), (2,)), ((0,), (0,))),
        preferred_element_type=jnp.float32)


def _to_row(v, eyeb):
    return jax.lax.dot_general(
        v, eyeb, (((1,), (1,)), ((0,), (0,))),
        preferred_element_type=jnp.float32)


def _iou(rc, cc):
    dx = jnp.clip(
        jnp.minimum(rc["maxx"], cc["maxx"]) - jnp.maximum(rc["minx"], cc["minx"]),
        0.0, None)
    dy = jnp.clip(
        jnp.minimum(rc["maxy"], cc["maxy"]) - jnp.maximum(rc["miny"], cc["miny"]),
        0.0, None)
    dz = jnp.clip(
        jnp.minimum(rc["maxz"], cc["maxz"]) - jnp.maximum(rc["minz"], cc["minz"]),
        0.0, None)
    inter = (dx * dy) * dz
    union = rc["vol"] + cc["vol"] - inter
    return inter / (union + 1e-6)


def _nms_kernel(reg_ref, prop_ref, s_ref, so_ref, bo_ref, a_scr):
    s = s_ref[...]

    def rg(k):
        return reg_ref[:, k:k + 1, :]

    def pp(k):
        return prop_ref[:, k:k + 1, :]

    pw = pp(3) + 0.05
    plen = pp(4) + 0.05
    ph = pp(5) + 0.05
    cx = rg(0) / 10.0 * pw + pp(0)
    cy = rg(1) / 10.0 * plen + pp(1)
    cz = rg(2) / 10.0 * ph + pp(2)
    w = jnp.exp(jnp.clip(rg(3) / 5.0, -4.0, 4.0)) * pw
    l = jnp.exp(jnp.clip(rg(4) / 5.0, -4.0, 4.0)) * plen
    h = jnp.exp(jnp.clip(rg(5) / 5.0, -4.0, 4.0)) * ph
    yaw = rg(6) + pp(6)

    bo_ref[...] = jnp.concatenate([cx, cy, cz, w, l, h, yaw], axis=1)

    row = {
        "minx": cx - w * 0.5, "miny": cy - l * 0.5, "minz": cz - h * 0.5,
        "maxx": cx + w * 0.5, "maxy": cy + l * 0.5, "maxz": cz + h * 0.5,
        "vol": (w * l) * h,
    }

    eyeb = _eyeb(_B)
    lane = jax.lax.broadcasted_iota(jnp.int32, (_NC, 1, _B), 2)

    kb_rows = []
    for b in range(_NB):
        sl = slice(b * _B, (b + 1) * _B)
        blk = {k: v[:, :, sl] for k, v in row.items()}
        blk_col = {k: _to_col(v, eyeb) for k, v in blk.items()}
        valid = (s[:, :, sl] > 0.0).astype(jnp.float32)

        if b > 0:
            prev = {k: v[:, :, : b * _B] for k, v in row.items()}
            keep_prev = jnp.concatenate(kb_rows, axis=2)
            iou_prev = _iou(blk_col, prev)
            ext_col = jnp.any(
                jnp.logical_and(iou_prev > _NMS_THRESH, keep_prev > 0.5),
                axis=2, keepdims=True)
            ext = _to_row(ext_col.astype(jnp.float32), eyeb)
            valid_eff = jnp.where(ext > 0.5, 0.0, valid)
        else:
            valid_eff = valid

        a_scr[...] = _iou(blk_col, blk)

        def body(i, kb, valid_eff=valid_eff):
            arow = a_scr[:, pl.ds(i, 1), :]
            sup = jnp.max(arow * kb, axis=2, keepdims=True)
            flag = (sup <= _NMS_THRESH).astype(jnp.float32)
            onehot = (lane == i).astype(jnp.float32)
            return kb + onehot * valid_eff * flag

        kb = jax.lax.fori_loop(
            0, _B, body, jnp.zeros((_NC, 1, _B), jnp.float32))
        kb_rows.append(kb)

    keep = jnp.concatenate(kb_rows, axis=2)
    so_ref[...] = jnp.where(keep > 0.5, s, -1.0)


@functools.partial(jax.jit, static_argnames=("interpret",))
def _run(class_logits, box_regression, proposals, interpret=False):
    n, c = class_logits.shape
    probs = jax.nn.softmax(class_logits, axis=-1)
    sj = jnp.where(probs[:, 1:] > _SCORE_THRESH, probs[:, 1:], -1.0).T
    top_s, top_i = jax.lax.top_k(sj, _PRE_NMS)

    reg3 = box_regression.reshape(n, c, 7)
    cls_ids = jnp.arange(1, c, dtype=jnp.int32)
    reg_cand = jax.vmap(lambda idx, j: reg3[idx, j])(top_i, cls_ids)
    prop_cand = proposals[top_i]

    pad = _M - _PRE_NMS
    s_p = jnp.pad(top_s, ((0, 0), (0, pad)), constant_values=-1.0)[:, None, :]
    regs_t = jnp.pad(reg_cand, ((0, 0), (0, pad), (0, 0))).transpose(0, 2, 1)
    props_t = jnp.pad(prop_cand, ((0, 0), (0, pad), (0, 0))).transpose(0, 2, 1)

    scores_o, boxes_o = pl.pallas_call(
        _nms_kernel,
        out_shape=[
            jax.ShapeDtypeStruct((_NC, 1, _M), jnp.float32),
            jax.ShapeDtypeStruct((_NC, 7, _M), jnp.float32),
        ],
        scratch_shapes=[pltpu.VMEM((_NC, _B, _B), jnp.float32)],
        interpret=interpret,
    )(regs_t, props_t, s_p)

    scores = scores_o[:, 0, :_PRE_NMS].reshape(-1)
    boxes = boxes_o.transpose(0, 2, 1)[:, :_PRE_NMS].reshape(-1, 7)
    labels = jnp.repeat(jnp.arange(1, c, dtype=jnp.int32), _PRE_NMS)

    fs, fi = jax.lax.top_k(scores, _DETS)
    fb = boxes[fi]
    fl = labels[fi]
    out = jnp.concatenate([fb, fs[:, None]], axis=1)
    return out, fl


def kernel(class_logits, box_regression, proposals):
    return _run(class_logits, box_regression, proposals)
